# baseline (speedup 1.0000x reference)
import functools

import jax
import jax.numpy as jnp
from jax import lax
from jax.experimental import pallas as pl
from jax.experimental.pallas import tpu as pltpu
from jax.experimental.pallas import tpu_sc as plsc

NC = 2
NS = 16
NW = NC * NS
L = 16

VOCAB = 1000000
D = 64
B = 4096
S = 200

CSEQ = 4
ROWS = CSEQ * S
IDXW = 100
NIDX = ROWS // IDXW
SEQ_PER_W = B // NW
NCHUNK = SEQ_PER_W // CSEQ


def _body(tok_hbm, table_hbm, pos_hbm, out_hbm, idx_v, rows_v, pos_v, sem):
    c = lax.axis_index("c")
    s = lax.axis_index("s")
    wid = s * NC + c

    pltpu.sync_copy(pos_hbm, pos_v)

    def chunk(g, carry):
        tok_row = wid * (SEQ_PER_W * S // IDXW) + g * NIDX
        out_row = wid * (SEQ_PER_W * S) + g * ROWS
        pltpu.sync_copy(tok_hbm.at[pl.ds(tok_row, NIDX)], idx_v)
        cps = [
            pltpu.async_copy(
                table_hbm.at[idx_v.at[j]],
                rows_v.at[pl.ds(j * IDXW, IDXW)],
                sem,
            )
            for j in range(NIDX)
        ]
        for cp in cps:
            cp.wait()

        def posloop(p, carry2):
            pv = [pos_v[p, pl.ds(d * L, L)] for d in range(D // L)]
            for cs in range(CSEQ):
                r = cs * S + p
                for d in range(D // L):
                    rows_v[r, pl.ds(d * L, L)] += pv[d]
            return carry2

        lax.fori_loop(0, S, posloop, 0, unroll=2)
        pltpu.sync_copy(rows_v, out_hbm.at[pl.ds(out_row, ROWS)])
        return carry

    lax.fori_loop(0, NCHUNK, chunk, 0)


@jax.jit
def _emb(tok, table, pos):
    mesh = plsc.VectorSubcoreMesh(
        core_axis_name="c", subcore_axis_name="s", num_cores=NC, num_subcores=NS
    )
    return pl.kernel(
        _body,
        out_type=jax.ShapeDtypeStruct((B * S, D), jnp.float32),
        mesh=mesh,
        scratch_types=[
            pltpu.VMEM((NIDX, IDXW), jnp.int32),
            pltpu.VMEM((ROWS, D), jnp.float32),
            pltpu.VMEM((S, D), jnp.float32),
            pltpu.SemaphoreType.DMA,
        ],
        compiler_params=pltpu.CompilerParams(use_tc_tiling_on_sc=False),
    )(tok, table, pos)


def kernel(tokens, embedding_table, pos_embedding_table):
    tok = tokens.astype(jnp.int32).reshape(B * S // IDXW, IDXW)
    out = _emb(tok, embedding_table, pos_embedding_table)
    return out.reshape(B, S, D)


# speedup vs baseline: 1.0807x; 1.0807x over previous
import jax
import jax.numpy as jnp
from jax import lax
from jax.experimental import pallas as pl
from jax.experimental.pallas import tpu as pltpu
from jax.experimental.pallas import tpu_sc as plsc

NC = 2
NS = 16
NW = NC * NS
L = 16

D = 64
B = 4096
S = 200

IDXW = 100
NIDX = S // IDXW
SEQ_PER_W = B // NW
NBUF = 4


def _body(tok_hbm, table_hbm, pos_hbm, out_hbm, idx_all, rows_v, pos_v, *sems):
    gsems = sems[:NBUF]
    osems = sems[NBUF:]
    c = lax.axis_index("c")
    s = lax.axis_index("s")
    wid = s * NC + c
    out_base = wid * (SEQ_PER_W * S)

    pltpu.sync_copy(tok_hbm.at[wid], idx_all)
    pltpu.sync_copy(pos_hbm, pos_v)

    def fire(g, b):
        for j in range(NIDX):
            pltpu.async_copy(
                table_hbm.at[idx_all.at[g, j]],
                rows_v.at[b, pl.ds(j * IDXW, IDXW)],
                gsems[b],
            )

    def drain_gather(g, b):
        for j in range(NIDX):
            pltpu.make_async_copy(
                table_hbm.at[idx_all.at[g, j]],
                rows_v.at[b, pl.ds(j * IDXW, IDXW)],
                gsems[b],
            ).wait()

    def pos_add(b):
        def posloop(p, carry):
            for d in range(D // L):
                plsc.addupdate(
                    rows_v.at[b, p, pl.ds(d * L, L)],
                    pos_v[p, pl.ds(d * L, L)],
                )
            return carry

        lax.fori_loop(0, S, posloop, 0, unroll=4)

    for b in range(NBUF - 1):
        fire(b, b)

    def outer(go, carry):
        for bi in range(NBUF):
            g = go * NBUF + bi
            drain_gather(g, bi)
            pos_add(bi)
            pltpu.async_copy(
                rows_v.at[bi],
                out_hbm.at[pl.ds(out_base + g * S, S)],
                osems[bi],
            )
            gp = g + NBUF - 1
            bp = (bi + NBUF - 1) % NBUF

            @pl.when(gp < SEQ_PER_W)
            def _():
                @pl.when(g >= 1)
                def _():
                    pltpu.make_async_copy(
                        rows_v.at[bp],
                        out_hbm.at[pl.ds(out_base + (g - 1) * S, S)],
                        osems[bp],
                    ).wait()

                fire(gp, bp)

        return carry

    lax.fori_loop(0, SEQ_PER_W // NBUF, outer, 0)

    for bi in range(NBUF):
        g = SEQ_PER_W - NBUF + bi
        pltpu.make_async_copy(
            rows_v.at[bi],
            out_hbm.at[pl.ds(out_base + g * S, S)],
            osems[bi],
        ).wait()


@jax.jit
def _emb(tok, table, pos):
    mesh = plsc.VectorSubcoreMesh(
        core_axis_name="c", subcore_axis_name="s", num_cores=NC, num_subcores=NS
    )
    return pl.kernel(
        _body,
        out_type=jax.ShapeDtypeStruct((B * S, D), jnp.float32),
        mesh=mesh,
        scratch_types=[
            pltpu.VMEM((SEQ_PER_W, NIDX, IDXW), jnp.int32),
            pltpu.VMEM((NBUF, S, D), jnp.float32),
            pltpu.VMEM((S, D), jnp.float32),
        ]
        + [pltpu.SemaphoreType.DMA] * (2 * NBUF),
        compiler_params=pltpu.CompilerParams(use_tc_tiling_on_sc=False),
    )(tok, table, pos)


def kernel(tokens, embedding_table, pos_embedding_table):
    tok = tokens.astype(jnp.int32).reshape(NW, SEQ_PER_W, NIDX, IDXW)
    out = _emb(tok, embedding_table, pos_embedding_table)
    return out.reshape(B, S, D)
